# Initial kernel scaffold; baseline (speedup 1.0000x reference)
import functools

import jax
import jax.numpy as jnp
from jax import lax
from jax.experimental import pallas as pl
from jax.experimental.pallas import tpu as pltpu
from jax.experimental.pallas import tpu_sc as plsc


def _build_sc_kernel(B, F, N, K, NC, NS):
    NW = NC * NS
    CB = 64
    IPC = CB * F
    assert IPC % 128 == 0
    JG = IPC // 128
    assert B % (NW * CB) == 0
    NCHUNK = B // (NW * CB)
    ROWS_W = B // NW

    mesh = plsc.VectorSubcoreMesh(core_axis_name="c", subcore_axis_name="s")

    @functools.partial(
        pl.kernel,
        out_type=jax.ShapeDtypeStruct((B, K), jnp.float32),
        mesh=mesh,
        scratch_types=[
            pltpu.VMEM((JG, 128), jnp.int32),
            pltpu.VMEM((IPC, K), jnp.float32),
            pltpu.VMEM((IPC, 1), jnp.float32),
            pltpu.VMEM((CB,), jnp.float32),
            pltpu.VMEM((CB, K), jnp.float32),
            pltpu.VMEM((1,), jnp.float32),
            pltpu.SemaphoreType.DMA,
        ],
    )
    def fm_kernel(idx_hbm, w0_hbm, w1_hbm, v_hbm, out_hbm,
                  idx_v, rows_v, w1_v, lin_v, out_v, w0_v, sem):
        cid = lax.axis_index("c")
        sid = lax.axis_index("s")
        wid = sid * NC + cid
        base_row = wid * ROWS_W

        pltpu.sync_copy(w0_hbm, w0_v)

        lane = lax.iota(jnp.int32, 16)
        zeros16 = jnp.zeros((16,), jnp.int32)

        def chunk_body(c, carry):
            row0 = base_row + c * CB
            q0 = row0 * F // 128

            pltpu.sync_copy(idx_hbm.at[pl.ds(q0, JG), :], idx_v)

            copies = []
            for j in range(JG):
                copies.append(pltpu.async_copy(
                    v_hbm.at[idx_v.at[j]],
                    rows_v.at[pl.ds(j * 128, 128), :], sem))
                copies.append(pltpu.async_copy(
                    w1_hbm.at[idx_v.at[j]],
                    w1_v.at[pl.ds(j * 128, 128), :], sem))
            for cp in copies:
                cp.wait()

            w0s = w0_v[0]
            for g in range(CB // 16):
                lin = jnp.full((16,), w0s, jnp.float32)
                for f in range(F):
                    r = lane * F + (g * 16 * F + f)
                    lin = lin + plsc.load_gather(w1_v, [r, zeros16])
                lin_v[pl.ds(g * 16, 16)] = lin

            def row_body(b, _):
                rbase = b * F
                x = rows_v[rbase]
                s = x
                sq = x * x
                for f in range(1, F):
                    x = rows_v[rbase + f]
                    s = s + x
                    sq = sq + x * x
                out_v[b] = 0.5 * (s * s - sq) + lin_v[b]
                return 0

            lax.fori_loop(0, CB, row_body, 0, unroll=2)

            pltpu.sync_copy(out_v, out_hbm.at[pl.ds(row0, CB), :])
            return carry

        lax.fori_loop(0, NCHUNK, chunk_body, 0)

    return fm_kernel


def kernel(inputs, W0, W1, V):
    B, F = inputs.shape
    N, K = V.shape
    info = plsc.get_sparse_core_info()
    NC, NS = info.num_cores, info.num_subcores
    idx2 = inputs.astype(jnp.int32).reshape(B * F // 128, 128)
    fm = _build_sc_kernel(B, F, N, K, NC, NS)
    return fm(idx2, W0, W1.astype(jnp.float32), V)


# speedup vs baseline: 1.2958x; 1.2958x over previous
import functools

import numpy as np
import jax
import jax.numpy as jnp
from jax import lax
from jax.experimental import pallas as pl
from jax.experimental.pallas import tpu as pltpu
from jax.experimental.pallas import tpu_sc as plsc


def _build_sc_kernel(B, F, N, K, NC, NS):
    NW = NC * NS
    CB = 64
    IPC = CB * F
    assert IPC % 128 == 0
    JG = IPC // 128
    assert B % (NW * CB) == 0
    NCHUNK = B // (NW * CB)
    ROWS_W = B // NW

    mesh = plsc.VectorSubcoreMesh(core_axis_name="c", subcore_axis_name="s")

    @functools.partial(
        pl.kernel,
        out_type=jax.ShapeDtypeStruct((B, K), jnp.float32),
        mesh=mesh,
        scratch_types=[
            pltpu.VMEM((ROWS_W * F // 128, 128), jnp.int32),
            pltpu.VMEM((IPC, K), jnp.float32),
            pltpu.VMEM((IPC + 16,), jnp.float32),
            pltpu.VMEM((CB, K), jnp.float32),
            pltpu.VMEM((16,), jnp.float32),
            pltpu.SemaphoreType.DMA,
        ],
        compiler_params=pltpu.CompilerParams(
            use_tc_tiling_on_sc=False, needs_layout_passes=False),
    )
    def fm_kernel(idx_hbm, w0_hbm, w1_hbm, v_hbm, out_hbm,
                  idx_v, rows_v, w1_v, out_v, w0_v, sem):
        cid = lax.axis_index("c")
        sid = lax.axis_index("s")
        wid = sid * NC + cid
        base_row = wid * ROWS_W

        pltpu.sync_copy(w0_hbm, w0_v)
        QW = ROWS_W * F // 128
        pltpu.sync_copy(idx_hbm.at[pl.ds(wid * QW, QW), :], idx_v)

        w0vec = w0_v[...]
        lane = lax.iota(jnp.int32, 16)
        fzero = jnp.zeros((16,), jnp.float32)

        def chunk_body(c, carry):
            row0 = base_row + c * CB

            copies = []
            for j in range(JG):
                copies.append(pltpu.async_copy(
                    v_hbm.at[idx_v.at[c * JG + j]],
                    rows_v.at[pl.ds(j * 128, 128), :], sem))
                copies.append(pltpu.async_copy(
                    w1_hbm.at[idx_v.at[c * JG + j]],
                    w1_v.at[pl.ds(j * 128, 128)], sem))
            for cp in copies:
                cp.wait()

            def row_body(b, _):
                rbase = b * F
                x = rows_v[rbase]
                s = x
                sq = x * x
                for f in range(1, F):
                    x = rows_v[rbase + f]
                    s = s + x
                    sq = sq + x * x
                wa = w1_v[pl.ds(rbase, 16)]
                wb = jnp.where(lane < (F - 16),
                               w1_v[pl.ds(rbase + 16, 16)], fzero)
                lin_b = jnp.sum(wa) + jnp.sum(wb)
                out_v[b] = 0.5 * (s * s - sq) + (lin_b + w0vec)
                return 0

            lax.fori_loop(0, CB, row_body, 0, unroll=2)

            pltpu.sync_copy(out_v, out_hbm.at[pl.ds(row0, CB), :])
            return carry

        lax.fori_loop(0, NCHUNK, chunk_body, 0)

    return fm_kernel


def kernel(inputs, W0, W1, V):
    B, F = inputs.shape
    N, K = V.shape
    info = plsc.get_sparse_core_info()
    NC, NS = info.num_cores, info.num_subcores
    idx2 = inputs.astype(jnp.int32).reshape(B * F // 128, 128)
    fm = _build_sc_kernel(B, F, N, K, NC, NS)
    w0_16 = jnp.broadcast_to(W0.astype(jnp.float32), (16,))
    w1_flat = W1.astype(jnp.float32).reshape(N)
    return fm(idx2, w0_16, w1_flat, V)
